# baseline (speedup 1.0000x reference)
import functools

import jax
import jax.numpy as jnp
from jax import lax
from jax.experimental import pallas as pl
from jax.experimental.pallas import tpu as pltpu
from jax.experimental.pallas import tpu_sc as plsc

NC, NS = 2, 16
NW = NC * NS
CH = 64


def _pe_table(S, D):
    pos = jnp.arange(S, dtype=jnp.float32)[:, None]
    div = jnp.exp(jnp.arange(0, D, 2, dtype=jnp.float32) * (-jnp.log(10000.0) / D))
    pe = jnp.zeros((S, D), dtype=jnp.float32)
    pe = pe.at[:, 0::2].set(jnp.sin(pos * div))
    pe = pe.at[:, 1::2].set(jnp.cos(pos * div))
    return pe


def _make_sc_kernel(B, S, V1):
    N = B * S
    NCHW = (N // CH) // NW
    mesh = plsc.VectorSubcoreMesh(
        core_axis_name="c", subcore_axis_name="s", num_cores=NC, num_subcores=NS
    )
    f32, i32 = jnp.float32, jnp.int32
    out_sds = jax.ShapeDtypeStruct((N, 128), f32)

    def phase_bufs():
        return (
            pltpu.VMEM((3 * CH,), f32),
            pltpu.VMEM((2 * CH,), f32),
            pltpu.VMEM((2 * CH,), i32),
            pltpu.VMEM((2 * CH,), f32),
            pltpu.VMEM((CH,), i32),
            pltpu.VMEM((CH,), i32),
            pltpu.VMEM((CH,), i32),
            pltpu.VMEM((CH, 64), f32),
            pltpu.VMEM((CH, 32), f32),
            pltpu.VMEM((CH, 32), f32),
            pltpu.VMEM((CH, 128), f32),
            pltpu.VMEM((CH, 128), f32),
            pltpu.VMEM((CH, 128), f32),
            pltpu.SemaphoreType.DMA,
            pltpu.SemaphoreType.DMA,
            pltpu.SemaphoreType.DMA,
        )

    @functools.partial(
        pl.kernel,
        out_type=(out_sds, out_sds, out_sds),
        mesh=mesh,
        scratch_types=[
            phase_bufs(), phase_bufs(),
            pltpu.VMEM((S, 128), f32),
            pltpu.VMEM((S, 128), f32),
            pltpu.VMEM((S, 128), f32),
            pltpu.VMEM((2, 32), f32),
            pltpu.VMEM((2, 32), f32),
            pltpu.VMEM((2, 64), f32),
        ],
        compiler_params=pltpu.CompilerParams(
            use_tc_tiling_on_sc=False, needs_layout_passes=False
        ),
    )
    def sc_kernel(
        src_hbm, dist_hbm, td_hbm, env_hbm, emb_src_hbm, emb_d_hbm,
        bpes_hbm, bped_hbm, bpee_hbm, ws_hbm, wd_hbm, we_hbm,
        o_src_hbm, o_dist_hbm, o_env_hbm,
        bufs0, bufs1, bpes_v, bped_v, bpee_v, ws_v, wd_v, we_v,
    ):
        w = lax.axis_index("s") * NC + lax.axis_index("c")
        bufs = (bufs0, bufs1)
        pltpu.sync_copy(bpes_hbm, bpes_v)
        pltpu.sync_copy(bped_hbm, bped_v)
        pltpu.sync_copy(bpee_hbm, bpee_v)
        pltpu.sync_copy(ws_hbm, ws_v)
        pltpu.sync_copy(wd_hbm, wd_v)
        pltpu.sync_copy(we_hbm, we_v)
        ws = [[ws_v[i, 16 * j:16 * (j + 1)] for j in range(2)] for i in range(2)]
        wd = [[wd_v[i, 16 * j:16 * (j + 1)] for j in range(2)] for i in range(2)]
        we = [[we_v[i, 16 * j:16 * (j + 1)] for j in range(4)] for i in range(2)]
        lanes = lax.iota(i32, 16)

        def base_of(c):
            return (w * NCHW + c) * CH

        def in_copies_desc(c, ph):
            base = base_of(c)
            sv, dv, tv, ev = bufs[ph][0], bufs[ph][1], bufs[ph][2], bufs[ph][3]
            sem = bufs[ph][13]
            return (
                pltpu.make_async_copy(src_hbm.at[pl.ds(3 * base, 3 * CH)], sv, sem),
                pltpu.make_async_copy(dist_hbm.at[pl.ds(2 * base, 2 * CH)], dv, sem),
                pltpu.make_async_copy(td_hbm.at[pl.ds(2 * base, 2 * CH)], tv, sem),
                pltpu.make_async_copy(env_hbm.at[pl.ds(2 * base, 2 * CH)], ev, sem),
            )

        def start_in(c, ph):
            for cp in in_copies_desc(c, ph):
                cp.start()

        def wait_in(c, ph):
            for cp in in_copies_desc(c, ph):
                cp.wait()

        def gather_desc(ph):
            idxs, td0b, td1b, g_src, g_d0, g_d1 = bufs[ph][4:10]
            sem = bufs[ph][14]
            return (
                pltpu.make_async_copy(emb_src_hbm.at[idxs], g_src, sem),
                pltpu.make_async_copy(emb_d_hbm.at[td0b], g_d0, sem),
                pltpu.make_async_copy(emb_d_hbm.at[td1b], g_d1, sem),
            )

        def build_idx_and_gather(ph):
            sv, tv = bufs[ph][0], bufs[ph][2]
            idxs, td0b, td1b = bufs[ph][4], bufs[ph][5], bufs[ph][6]
            for m in range(CH // 16):
                rows = 16 * m + lanes
                fv = plsc.load_gather(sv, [3 * rows + 2])
                idxs[pl.ds(16 * m, 16)] = fv.astype(i32)
                td0b[pl.ds(16 * m, 16)] = plsc.load_gather(tv, [2 * rows])
                td1b[pl.ds(16 * m, 16)] = plsc.load_gather(tv, [2 * rows + 1]) + V1
            for cp in gather_desc(ph):
                cp.start()

        def wait_gathers(ph):
            for cp in gather_desc(ph):
                cp.wait()

        def out_desc(c, ph):
            base = base_of(c)
            o_src, o_dist, o_env = bufs[ph][10], bufs[ph][11], bufs[ph][12]
            sem = bufs[ph][15]
            return (
                pltpu.make_async_copy(o_src, o_src_hbm.at[pl.ds(base, CH)], sem),
                pltpu.make_async_copy(o_dist, o_dist_hbm.at[pl.ds(base, CH)], sem),
                pltpu.make_async_copy(o_env, o_env_hbm.at[pl.ds(base, CH)], sem),
            )

        def compute(c, ph):
            base = base_of(c)
            sv, dv, ev = bufs[ph][0], bufs[ph][1], bufs[ph][3]
            g_src, g_d0, g_d1 = bufs[ph][7], bufs[ph][8], bufs[ph][9]
            o_src, o_dist, o_env = bufs[ph][10], bufs[ph][11], bufs[ph][12]

            @plsc.parallel_loop(0, CH // 16, 1, unroll=2)
            def grp(g):
                tb = 16 * g
                rows = tb + lanes
                x0v = plsc.load_gather(sv, [3 * rows])
                x1v = plsc.load_gather(sv, [3 * rows + 1])
                d0v = plsc.load_gather(dv, [2 * rows])
                d1v = plsc.load_gather(dv, [2 * rows + 1])
                e0v = plsc.load_gather(ev, [2 * rows])
                e1v = plsc.load_gather(ev, [2 * rows + 1])
                for k in range(16):
                    t = tb + k
                    s = lax.rem(base + t, S)
                    x0, x1 = x0v[k], x1v[k]
                    d0, d1 = d0v[k], d1v[k]
                    e0, e1 = e0v[k], e1v[k]
                    for j in range(2):
                        a, b = 16 * j, 16 * (j + 1)
                        o_src[t, a:b] = x0 * ws[0][j] + bpes_v[s, a:b]
                        o_src[t, 32 + a:32 + b] = x1 * ws[1][j] + bpes_v[s, 32 + a:32 + b]
                        o_dist[t, a:b] = g_d0[t, a:b] + bped_v[s, a:b]
                        o_dist[t, 32 + a:32 + b] = d0 * wd[0][j] + bped_v[s, 32 + a:32 + b]
                        o_dist[t, 64 + a:64 + b] = g_d1[t, a:b] + bped_v[s, 64 + a:64 + b]
                        o_dist[t, 96 + a:96 + b] = d1 * wd[1][j] + bped_v[s, 96 + a:96 + b]
                    for j in range(4):
                        a, b = 16 * j, 16 * (j + 1)
                        o_src[t, 64 + a:64 + b] = g_src[t, a:b] + bpes_v[s, 64 + a:64 + b]
                        o_env[t, a:b] = e0 * we[0][j] + bpee_v[s, a:b]
                        o_env[t, 64 + a:64 + b] = e1 * we[1][j] + bpee_v[s, 64 + a:64 + b]

        start_in(0, 0)
        start_in(1, 1)
        wait_in(0, 0)
        build_idx_and_gather(0)

        def pair_body(p, carry):
            for ph in (0, 1):
                c = 2 * p + ph
                nxt = ph ^ 1

                @pl.when(c + 1 < NCHW)
                def _():
                    wait_in(c + 1, nxt)
                    build_idx_and_gather(nxt)

                wait_gathers(ph)

                @pl.when(c >= 2)
                def _():
                    for cp in out_desc(c - 2, ph):
                        cp.wait()

                compute(c, ph)
                for cp in out_desc(c, ph):
                    cp.start()

                @pl.when(c + 2 < NCHW)
                def _():
                    start_in(c + 2, ph)
            return carry

        lax.fori_loop(0, NCHW // 2, pair_body, 0)
        for cp in out_desc(NCHW - 2, 0):
            cp.wait()
        for cp in out_desc(NCHW - 1, 1):
            cp.wait()

    return sc_kernel


def kernel(src, dist, type_dist, env_dist, W_src, b_src, emb_src,
           W_dist, b_dist, emb_dist, W_env, b_env):
    B, S, _ = src.shape
    V1 = emb_dist.shape[1]

    src1 = src.reshape(B * S * 3)
    dist1 = dist.reshape(B * S * 2)
    td1 = type_dist.reshape(B * S * 2).astype(jnp.int32)
    env1 = env_dist.reshape(B * S * 2)
    emb_d = emb_dist.reshape(2 * V1, emb_dist.shape[2])

    pe = _pe_table(S, 128)
    bpe_src = jnp.concatenate(
        [b_src[0][None, :] + pe[:, 0:32],
         b_src[1][None, :] + pe[:, 32:64],
         pe[:, 64:128]], axis=1)
    bpe_dist = jnp.concatenate(
        [pe[:, 0:32],
         b_dist[0][None, :] + pe[:, 32:64],
         pe[:, 64:96],
         b_dist[1][None, :] + pe[:, 96:128]], axis=1)
    bpe_env = jnp.concatenate(
        [b_env[0][None, :] + pe[:, 0:64],
         b_env[1][None, :] + pe[:, 64:128]], axis=1)

    sc = _make_sc_kernel(B, S, V1)
    o_src, o_dist, o_env = sc(src1, dist1, td1, env1, emb_src, emb_d,
                              bpe_src, bpe_dist, bpe_env, W_src, W_dist, W_env)
    return (o_src.reshape(B, S, 128),
            o_dist.reshape(B, S, 128),
            o_env.reshape(B, S, 128))


# speedup vs baseline: 1.5868x; 1.5868x over previous
import functools

import jax
import jax.numpy as jnp
from jax import lax
from jax.experimental import pallas as pl
from jax.experimental.pallas import tpu as pltpu
from jax.experimental.pallas import tpu_sc as plsc

NC, NS = 2, 16
NW = NC * NS
CH = 128


def _pe_table(S, D):
    pos = jnp.arange(S, dtype=jnp.float32)[:, None]
    div = jnp.exp(jnp.arange(0, D, 2, dtype=jnp.float32) * (-jnp.log(10000.0) / D))
    pe = jnp.zeros((S, D), dtype=jnp.float32)
    pe = pe.at[:, 0::2].set(jnp.sin(pos * div))
    pe = pe.at[:, 1::2].set(jnp.cos(pos * div))
    return pe


def _make_idx_kernel(B, S, V1, BB):
    N = B * S
    i32 = jnp.int32

    def body(src_ref, td_ref, i0_ref, i1_ref, i2_ref):
        i0_ref[...] = src_ref[:, :, 2].astype(i32).reshape(BB * S)
        i1_ref[...] = td_ref[:, :, 0].reshape(BB * S)
        i2_ref[...] = td_ref[:, :, 1].reshape(BB * S) + V1

    grid = B // BB
    return pl.pallas_call(
        body,
        grid=(grid,),
        in_specs=[
            pl.BlockSpec((BB, S, 3), lambda c: (c, 0, 0)),
            pl.BlockSpec((BB, S, 2), lambda c: (c, 0, 0)),
        ],
        out_specs=[
            pl.BlockSpec((BB * S,), lambda c: (c,)),
            pl.BlockSpec((BB * S,), lambda c: (c,)),
            pl.BlockSpec((BB * S,), lambda c: (c,)),
        ],
        out_shape=[jax.ShapeDtypeStruct((N,), i32)] * 3,
    )


def _make_sc_gather(N, V0, V1D, V1):
    NCHW = (N // CH) // NW
    mesh = plsc.VectorSubcoreMesh(
        core_axis_name="c", subcore_axis_name="s", num_cores=NC, num_subcores=NS
    )
    f32, i32 = jnp.float32, jnp.int32

    def phase_bufs():
        return (
            pltpu.VMEM((CH,), i32),
            pltpu.VMEM((CH,), i32),
            pltpu.VMEM((CH,), i32),
            pltpu.VMEM((CH, 64), f32),
            pltpu.VMEM((CH, 32), f32),
            pltpu.VMEM((CH, 32), f32),
            pltpu.SemaphoreType.DMA,
            pltpu.SemaphoreType.DMA,
            pltpu.SemaphoreType.DMA,
        )

    @functools.partial(
        pl.kernel,
        out_type=jax.ShapeDtypeStruct((N, 128), f32),
        mesh=mesh,
        scratch_types=[phase_bufs(), phase_bufs()],
        compiler_params=pltpu.CompilerParams(
            use_tc_tiling_on_sc=False, needs_layout_passes=False
        ),
    )
    def sc_kernel(i0_hbm, i1_hbm, i2_hbm, emb_src_hbm, emb_d_hbm,
                  gat_hbm, bufs0, bufs1):
        w = lax.axis_index("s") * NC + lax.axis_index("c")
        bufs = (bufs0, bufs1)

        def base_of(c):
            return (w * NCHW + c) * CH

        def in_desc(c, ph):
            base = base_of(c)
            v0, v1, v2 = bufs[ph][0], bufs[ph][1], bufs[ph][2]
            sem = bufs[ph][6]
            return (
                pltpu.make_async_copy(i0_hbm.at[pl.ds(base, CH)], v0, sem),
                pltpu.make_async_copy(i1_hbm.at[pl.ds(base, CH)], v1, sem),
                pltpu.make_async_copy(i2_hbm.at[pl.ds(base, CH)], v2, sem),
            )

        def gather_desc(ph):
            v0, v1, v2, g0, g1, g2 = bufs[ph][:6]
            sem = bufs[ph][7]
            return (
                pltpu.make_async_copy(emb_src_hbm.at[v0], g0, sem),
                pltpu.make_async_copy(emb_d_hbm.at[v1], g1, sem),
                pltpu.make_async_copy(emb_d_hbm.at[v2], g2, sem),
            )

        def out_desc(c, ph):
            base = base_of(c)
            g0, g1, g2 = bufs[ph][3], bufs[ph][4], bufs[ph][5]
            sem = bufs[ph][8]
            rows = gat_hbm.at[pl.ds(base, CH)]
            return (
                pltpu.make_async_copy(g0, rows.at[:, pl.ds(0, 64)], sem),
                pltpu.make_async_copy(g1, rows.at[:, pl.ds(64, 32)], sem),
                pltpu.make_async_copy(g2, rows.at[:, pl.ds(96, 32)], sem),
            )

        def start(ds):
            for d in ds:
                d.start()

        def wait(ds):
            for d in ds:
                d.wait()

        start(in_desc(0, 0))
        start(in_desc(1, 1))
        wait(in_desc(0, 0))
        start(gather_desc(0))

        def pair_body(p, carry):
            for ph in (0, 1):
                c = 2 * p + ph
                nxt = ph ^ 1
                wait(gather_desc(ph))
                start(out_desc(c, ph))

                @pl.when(c + 1 < NCHW)
                def _():
                    wait(in_desc(c + 1, nxt))

                    @pl.when(c + 2 < NCHW)
                    def _():
                        start(in_desc(c + 2, ph))

                    @pl.when(c >= 1)
                    def _():
                        wait(out_desc(c - 1, nxt))

                    start(gather_desc(nxt))
            return carry

        lax.fori_loop(0, NCHW // 2, pair_body, 0)
        wait(out_desc(NCHW - 2, 0))
        wait(out_desc(NCHW - 1, 1))

    return sc_kernel


def _make_asm_kernel(B, S, BB):
    f32 = jnp.float32

    def body(src_ref, dist_ref, env_ref, gat_ref,
             bpes_ref, bped_ref, bpee_ref, ws_ref, wd_ref, we_ref,
             osrc_ref, odist_ref, oenv_ref):
        bs = bpes_ref[...][None, :, :]
        bd = bped_ref[...][None, :, :]
        be = bpee_ref[...][None, :, :]
        x0 = src_ref[:, :, 0:1]
        x1 = src_ref[:, :, 1:2]
        d0 = dist_ref[:, :, 0:1]
        d1 = dist_ref[:, :, 1:2]
        e0 = env_ref[:, :, 0:1]
        e1 = env_ref[:, :, 1:2]
        gat = gat_ref[...]
        osrc_ref[:, :, 0:32] = x0 * ws_ref[0][None, None, :] + bs[:, :, 0:32]
        osrc_ref[:, :, 32:64] = x1 * ws_ref[1][None, None, :] + bs[:, :, 32:64]
        osrc_ref[:, :, 64:128] = gat[:, :, 0:64] + bs[:, :, 64:128]
        odist_ref[:, :, 0:32] = gat[:, :, 64:96] + bd[:, :, 0:32]
        odist_ref[:, :, 32:64] = d0 * wd_ref[0][None, None, :] + bd[:, :, 32:64]
        odist_ref[:, :, 64:96] = gat[:, :, 96:128] + bd[:, :, 64:96]
        odist_ref[:, :, 96:128] = d1 * wd_ref[1][None, None, :] + bd[:, :, 96:128]
        oenv_ref[:, :, 0:64] = e0 * we_ref[0][None, None, :] + be[:, :, 0:64]
        oenv_ref[:, :, 64:128] = e1 * we_ref[1][None, None, :] + be[:, :, 64:128]

    grid = B // BB
    full = lambda shape: pl.BlockSpec(shape, lambda c: tuple(0 for _ in shape))
    blk3 = lambda m: pl.BlockSpec((BB, S, m), lambda c: (c, 0, 0))
    return pl.pallas_call(
        body,
        grid=(grid,),
        in_specs=[
            blk3(3), blk3(2), blk3(2), blk3(128),
            full((S, 128)), full((S, 128)), full((S, 128)),
            full((2, 32)), full((2, 32)), full((2, 64)),
        ],
        out_specs=[blk3(128), blk3(128), blk3(128)],
        out_shape=[jax.ShapeDtypeStruct((B, S, 128), f32)] * 3,
    )


def kernel(src, dist, type_dist, env_dist, W_src, b_src, emb_src,
           W_dist, b_dist, emb_dist, W_env, b_env):
    B, S, _ = src.shape
    N = B * S
    V1 = emb_dist.shape[1]
    BB = 64

    td = type_dist.astype(jnp.int32)
    emb_d = emb_dist.reshape(2 * V1, emb_dist.shape[2])

    pe = _pe_table(S, 128)
    bpe_src = jnp.concatenate(
        [b_src[0][None, :] + pe[:, 0:32],
         b_src[1][None, :] + pe[:, 32:64],
         pe[:, 64:128]], axis=1)
    bpe_dist = jnp.concatenate(
        [pe[:, 0:32],
         b_dist[0][None, :] + pe[:, 32:64],
         pe[:, 64:96],
         b_dist[1][None, :] + pe[:, 96:128]], axis=1)
    bpe_env = jnp.concatenate(
        [b_env[0][None, :] + pe[:, 0:64],
         b_env[1][None, :] + pe[:, 64:128]], axis=1)

    i0, i1, i2 = _make_idx_kernel(B, S, V1, 512)(src, td)
    gat = _make_sc_gather(N, emb_src.shape[0], 2 * V1, V1)(
        i0, i1, i2, emb_src, emb_d)
    gat3 = gat.reshape(B, S, 128)
    o_src, o_dist, o_env = _make_asm_kernel(B, S, BB)(
        src, dist, env_dist, gat3,
        bpe_src, bpe_dist, bpe_env, W_src, W_dist, W_env)
    return (o_src, o_dist, o_env)


# speedup vs baseline: 1.6019x; 1.0095x over previous
import functools

import jax
import jax.numpy as jnp
from jax import lax
from jax.experimental import pallas as pl
from jax.experimental.pallas import tpu as pltpu
from jax.experimental.pallas import tpu_sc as plsc

NC, NS = 2, 16
NW = NC * NS
CH = 128


def _pe_table(S, D):
    pos = jnp.arange(S, dtype=jnp.float32)[:, None]
    div = jnp.exp(jnp.arange(0, D, 2, dtype=jnp.float32) * (-jnp.log(10000.0) / D))
    pe = jnp.zeros((S, D), dtype=jnp.float32)
    pe = pe.at[:, 0::2].set(jnp.sin(pos * div))
    pe = pe.at[:, 1::2].set(jnp.cos(pos * div))
    return pe


def _make_idx_kernel(B, S, V1, BB):
    N = B * S
    i32 = jnp.int32

    def body(src_ref, td_ref, i0_ref, i1_ref, i2_ref):
        i0_ref[...] = src_ref[:, :, 2].astype(i32).reshape(BB * S)
        i1_ref[...] = td_ref[:, :, 0].reshape(BB * S)
        i2_ref[...] = td_ref[:, :, 1].reshape(BB * S)

    grid = B // BB
    return pl.pallas_call(
        body,
        grid=(grid,),
        in_specs=[
            pl.BlockSpec((BB, S, 3), lambda c: (c, 0, 0)),
            pl.BlockSpec((BB, S, 2), lambda c: (c, 0, 0)),
        ],
        out_specs=[
            pl.BlockSpec((BB * S,), lambda c: (c,)),
            pl.BlockSpec((BB * S,), lambda c: (c,)),
            pl.BlockSpec((BB * S,), lambda c: (c,)),
        ],
        out_shape=[jax.ShapeDtypeStruct((N,), i32)] * 3,
    )


def _make_sc_gather(N, V0, V1D, V1):
    NCHW = (N // CH) // NW
    mesh = plsc.VectorSubcoreMesh(
        core_axis_name="c", subcore_axis_name="s", num_cores=NC, num_subcores=NS
    )
    f32, i32 = jnp.float32, jnp.int32

    def phase_bufs():
        return (
            pltpu.VMEM((CH,), i32),
            pltpu.VMEM((CH,), i32),
            pltpu.VMEM((CH,), i32),
            pltpu.VMEM((CH, 64), f32),
            pltpu.VMEM((CH, 32), f32),
            pltpu.VMEM((CH, 32), f32),
            pltpu.SemaphoreType.DMA,
            pltpu.SemaphoreType.DMA,
            pltpu.SemaphoreType.DMA,
        )

    @functools.partial(
        pl.kernel,
        out_type=jax.ShapeDtypeStruct((N, 128), f32),
        mesh=mesh,
        scratch_types=[phase_bufs(), phase_bufs()],
        compiler_params=pltpu.CompilerParams(
            use_tc_tiling_on_sc=False, needs_layout_passes=False
        ),
    )
    def sc_kernel(i0_hbm, i1_hbm, i2_hbm, emb_src_hbm, emb_dist_hbm,
                  gat_hbm, bufs0, bufs1):
        emb_d0_hbm = emb_dist_hbm.at[0]
        emb_d1_hbm = emb_dist_hbm.at[1]
        w = lax.axis_index("s") * NC + lax.axis_index("c")
        bufs = (bufs0, bufs1)

        def base_of(c):
            return (w * NCHW + c) * CH

        def in_desc(c, ph):
            base = base_of(c)
            v0, v1, v2 = bufs[ph][0], bufs[ph][1], bufs[ph][2]
            sem = bufs[ph][6]
            return (
                pltpu.make_async_copy(i0_hbm.at[pl.ds(base, CH)], v0, sem),
                pltpu.make_async_copy(i1_hbm.at[pl.ds(base, CH)], v1, sem),
                pltpu.make_async_copy(i2_hbm.at[pl.ds(base, CH)], v2, sem),
            )

        def gather_desc(ph):
            v0, v1, v2, g0, g1, g2 = bufs[ph][:6]
            sem = bufs[ph][7]
            return (
                pltpu.make_async_copy(emb_src_hbm.at[v0], g0, sem),
                pltpu.make_async_copy(emb_d0_hbm.at[v1], g1, sem),
                pltpu.make_async_copy(emb_d1_hbm.at[v2], g2, sem),
            )

        def out_desc(c, ph):
            base = base_of(c)
            g0, g1, g2 = bufs[ph][3], bufs[ph][4], bufs[ph][5]
            sem = bufs[ph][8]
            rows = gat_hbm.at[pl.ds(base, CH)]
            return (
                pltpu.make_async_copy(g0, rows.at[:, pl.ds(0, 64)], sem),
                pltpu.make_async_copy(g1, rows.at[:, pl.ds(64, 32)], sem),
                pltpu.make_async_copy(g2, rows.at[:, pl.ds(96, 32)], sem),
            )

        def start(ds):
            for d in ds:
                d.start()

        def wait(ds):
            for d in ds:
                d.wait()

        start(in_desc(0, 0))
        start(in_desc(1, 1))
        wait(in_desc(0, 0))
        start(gather_desc(0))

        def pair_body(p, carry):
            for ph in (0, 1):
                c = 2 * p + ph
                nxt = ph ^ 1
                wait(gather_desc(ph))
                start(out_desc(c, ph))

                @pl.when(c + 1 < NCHW)
                def _():
                    wait(in_desc(c + 1, nxt))

                    @pl.when(c + 2 < NCHW)
                    def _():
                        start(in_desc(c + 2, ph))

                    @pl.when(c >= 1)
                    def _():
                        wait(out_desc(c - 1, nxt))

                    start(gather_desc(nxt))
            return carry

        lax.fori_loop(0, NCHW // 2, pair_body, 0)
        wait(out_desc(NCHW - 2, 0))
        wait(out_desc(NCHW - 1, 1))

    return sc_kernel


def _make_asm_kernel(B, S, BB):
    f32 = jnp.float32

    def body(src_ref, dist_ref, env_ref, gat_ref,
             bpes_ref, bped_ref, bpee_ref, ws_ref, wd_ref, we_ref,
             osrc_ref, odist_ref, oenv_ref):
        bs = bpes_ref[...][None, :, :]
        bd = bped_ref[...][None, :, :]
        be = bpee_ref[...][None, :, :]
        x0 = src_ref[:, :, 0:1]
        x1 = src_ref[:, :, 1:2]
        d0 = dist_ref[:, :, 0:1]
        d1 = dist_ref[:, :, 1:2]
        e0 = env_ref[:, :, 0:1]
        e1 = env_ref[:, :, 1:2]
        gat = gat_ref[...]
        osrc_ref[:, :, 0:32] = x0 * ws_ref[0][None, None, :] + bs[:, :, 0:32]
        osrc_ref[:, :, 32:64] = x1 * ws_ref[1][None, None, :] + bs[:, :, 32:64]
        osrc_ref[:, :, 64:128] = gat[:, :, 0:64] + bs[:, :, 64:128]
        odist_ref[:, :, 0:32] = gat[:, :, 64:96] + bd[:, :, 0:32]
        odist_ref[:, :, 32:64] = d0 * wd_ref[0][None, None, :] + bd[:, :, 32:64]
        odist_ref[:, :, 64:96] = gat[:, :, 96:128] + bd[:, :, 64:96]
        odist_ref[:, :, 96:128] = d1 * wd_ref[1][None, None, :] + bd[:, :, 96:128]
        oenv_ref[:, :, 0:64] = e0 * we_ref[0][None, None, :] + be[:, :, 0:64]
        oenv_ref[:, :, 64:128] = e1 * we_ref[1][None, None, :] + be[:, :, 64:128]

    grid = B // BB
    full = lambda shape: pl.BlockSpec(shape, lambda c: tuple(0 for _ in shape))
    blk3 = lambda m: pl.BlockSpec((BB, S, m), lambda c: (c, 0, 0))
    return pl.pallas_call(
        body,
        grid=(grid,),
        in_specs=[
            blk3(3), blk3(2), blk3(2), blk3(128),
            full((S, 128)), full((S, 128)), full((S, 128)),
            full((2, 32)), full((2, 32)), full((2, 64)),
        ],
        out_specs=[blk3(128), blk3(128), blk3(128)],
        out_shape=[jax.ShapeDtypeStruct((B, S, 128), f32)] * 3,
    )


def kernel(src, dist, type_dist, env_dist, W_src, b_src, emb_src,
           W_dist, b_dist, emb_dist, W_env, b_env):
    B, S, _ = src.shape
    N = B * S
    V1 = emb_dist.shape[1]
    BB = 64

    td = type_dist.astype(jnp.int32)

    pe = _pe_table(S, 128)
    bpe_src = jnp.concatenate(
        [b_src[0][None, :] + pe[:, 0:32],
         b_src[1][None, :] + pe[:, 32:64],
         pe[:, 64:128]], axis=1)
    bpe_dist = jnp.concatenate(
        [pe[:, 0:32],
         b_dist[0][None, :] + pe[:, 32:64],
         pe[:, 64:96],
         b_dist[1][None, :] + pe[:, 96:128]], axis=1)
    bpe_env = jnp.concatenate(
        [b_env[0][None, :] + pe[:, 0:64],
         b_env[1][None, :] + pe[:, 64:128]], axis=1)

    i0, i1, i2 = _make_idx_kernel(B, S, V1, 512)(src, td)
    gat = _make_sc_gather(N, emb_src.shape[0], 2 * V1, V1)(
        i0, i1, i2, emb_src, emb_dist)
    gat3 = gat.reshape(B, S, 128)
    o_src, o_dist, o_env = _make_asm_kernel(B, S, BB)(
        src, dist, env_dist, gat3,
        bpe_src, bpe_dist, bpe_env, W_src, W_dist, W_env)
    return (o_src, o_dist, o_env)
